# baseline (speedup 1.0000x reference)
import functools

import jax
import jax.numpy as jnp
from jax.experimental import pallas as pl
from jax.experimental.pallas import tpu as pltpu


def _lin1_body(x_ref, w_ref, ms_ref, md_ref, h_ref, asrc_ref, adst_ref):
    h = jnp.dot(x_ref[:], w_ref[:], preferred_element_type=jnp.float32)
    h_ref[:] = h
    asrc_ref[:] = jnp.dot(h, ms_ref[:], preferred_element_type=jnp.float32)
    adst_ref[:] = jnp.dot(h, md_ref[:], preferred_element_type=jnp.float32)


def _edge1_body(src_ref, dst_ref, h_ref, asrc_ref, adst_ref, sel_ref,
                acc_ref, den_ref, *, blk):
    pid = pl.program_id(0)

    @pl.when(pid == 0)
    def _():
        acc_ref[:] = jnp.zeros_like(acc_ref)
        den_ref[:] = jnp.zeros_like(den_ref)

    def body(j, carry):
        s = src_ref[0, 0, j]
        d = dst_ref[0, 0, j]
        v = asrc_ref[pl.ds(s, 1), :] + adst_ref[pl.ds(d, 1), :]
        v = jnp.where(v > 0, v, 0.2 * v)
        w = jnp.exp(v)
        den_ref[pl.ds(d, 1), :] += w
        wf = jnp.dot(w, sel_ref[:], preferred_element_type=jnp.float32)
        acc_ref[pl.ds(d, 1), :] += h_ref[pl.ds(s, 1), :] * wf
        return carry

    jax.lax.fori_loop(0, blk, body, 0)


def _norm1_body(acc_ref, den_ref, b_ref, sel_ref, o_ref):
    denf = jnp.dot(den_ref[:], sel_ref[:], preferred_element_type=jnp.float32)
    z = acc_ref[:] / denf + b_ref[:]
    o_ref[:] = jnp.where(z > 0, z, jnp.exp(z) - 1.0)


def _lin2_body(h_ref, w_ref, as_ref, ad_ref, h2_ref, asrc_ref, adst_ref):
    h2 = jnp.dot(h_ref[:], w_ref[:], preferred_element_type=jnp.float32)
    h2_ref[:] = h2
    asrc_ref[:] = jnp.dot(h2, as_ref[:], preferred_element_type=jnp.float32)
    adst_ref[:] = jnp.dot(h2, ad_ref[:], preferred_element_type=jnp.float32)


def _edge2_body(src_ref, dst_ref, h_ref, asrc_ref, adst_ref, b_ref, o_ref,
                acc_ref, den_ref, *, blk, nblk):
    pid = pl.program_id(0)

    @pl.when(pid == 0)
    def _():
        acc_ref[:] = jnp.zeros_like(acc_ref)
        den_ref[:] = jnp.zeros_like(den_ref)

    def body(j, carry):
        s = src_ref[0, 0, j]
        d = dst_ref[0, 0, j]
        v = asrc_ref[pl.ds(s, 1), :] + adst_ref[pl.ds(d, 1), :]
        v = jnp.where(v > 0, v, 0.2 * v)
        w = jnp.exp(v)
        den_ref[pl.ds(d, 1), :] += w
        acc_ref[pl.ds(d, 1), :] += h_ref[pl.ds(s, 1), :] * w
        return carry

    jax.lax.fori_loop(0, blk, body, 0)

    @pl.when(pid == nblk - 1)
    def _():
        o_ref[:] = acc_ref[:] / den_ref[:] + b_ref[:]


def kernel(x, edge_index, W1, a_src1, a_dst1, b1, W2, a_src2, a_dst2, b2):
    n, d_in = x.shape
    e = edge_index.shape[1]
    heads, ch = a_src1.shape[1], a_src1.shape[2]
    hc = heads * ch
    out_dim = W2.shape[1]
    f32 = jnp.float32

    loops = jnp.arange(n, dtype=edge_index.dtype)
    ei = jnp.concatenate([edge_index, jnp.stack([loops, loops])], axis=1)
    et = e + n
    blk = next(b for b in range(2048, 0, -1) if et % b == 0)
    nblk = et // blk
    src3 = ei[0].reshape(nblk, 1, blk)
    dst3 = ei[1].reshape(nblk, 1, blk)

    rows = jnp.arange(hc)
    cols = rows // ch
    ms1 = jnp.zeros((hc, heads), f32).at[rows, cols].set(a_src1.reshape(-1))
    md1 = jnp.zeros((hc, heads), f32).at[rows, cols].set(a_dst1.reshape(-1))
    sel = jnp.zeros((heads, hc), f32).at[cols, rows].set(1.0)
    a2s = a_src2.reshape(out_dim, 1).astype(f32)
    a2d = a_dst2.reshape(out_dim, 1).astype(f32)

    rblk = 1000 if n % 1000 == 0 else n
    nrb = n // rblk

    h1, asrc1, adst1 = pl.pallas_call(
        _lin1_body,
        grid=(nrb,),
        in_specs=[
            pl.BlockSpec((rblk, d_in), lambda i: (i, 0)),
            pl.BlockSpec((d_in, hc), lambda i: (0, 0)),
            pl.BlockSpec((hc, heads), lambda i: (0, 0)),
            pl.BlockSpec((hc, heads), lambda i: (0, 0)),
        ],
        out_specs=[
            pl.BlockSpec((rblk, hc), lambda i: (i, 0)),
            pl.BlockSpec((rblk, heads), lambda i: (i, 0)),
            pl.BlockSpec((rblk, heads), lambda i: (i, 0)),
        ],
        out_shape=[
            jax.ShapeDtypeStruct((n, hc), f32),
            jax.ShapeDtypeStruct((n, heads), f32),
            jax.ShapeDtypeStruct((n, heads), f32),
        ],
    )(x, W1, ms1, md1)

    acc1, den1 = pl.pallas_call(
        functools.partial(_edge1_body, blk=blk),
        grid=(nblk,),
        in_specs=[
            pl.BlockSpec((1, 1, blk), lambda i: (i, 0, 0),
                         memory_space=pltpu.SMEM),
            pl.BlockSpec((1, 1, blk), lambda i: (i, 0, 0),
                         memory_space=pltpu.SMEM),
            pl.BlockSpec((n, hc), lambda i: (0, 0)),
            pl.BlockSpec((n, heads), lambda i: (0, 0)),
            pl.BlockSpec((n, heads), lambda i: (0, 0)),
            pl.BlockSpec((heads, hc), lambda i: (0, 0)),
        ],
        out_specs=[
            pl.BlockSpec((n, hc), lambda i: (0, 0)),
            pl.BlockSpec((n, heads), lambda i: (0, 0)),
        ],
        out_shape=[
            jax.ShapeDtypeStruct((n, hc), f32),
            jax.ShapeDtypeStruct((n, heads), f32),
        ],
    )(src3, dst3, h1, asrc1, adst1, sel)

    h1o = pl.pallas_call(
        _norm1_body,
        grid=(nrb,),
        in_specs=[
            pl.BlockSpec((rblk, hc), lambda i: (i, 0)),
            pl.BlockSpec((rblk, heads), lambda i: (i, 0)),
            pl.BlockSpec((1, hc), lambda i: (0, 0)),
            pl.BlockSpec((heads, hc), lambda i: (0, 0)),
        ],
        out_specs=pl.BlockSpec((rblk, hc), lambda i: (i, 0)),
        out_shape=jax.ShapeDtypeStruct((n, hc), f32),
    )(acc1, den1, b1.reshape(1, hc).astype(f32), sel)

    h2, asrc2, adst2 = pl.pallas_call(
        _lin2_body,
        grid=(nrb,),
        in_specs=[
            pl.BlockSpec((rblk, hc), lambda i: (i, 0)),
            pl.BlockSpec((hc, out_dim), lambda i: (0, 0)),
            pl.BlockSpec((out_dim, 1), lambda i: (0, 0)),
            pl.BlockSpec((out_dim, 1), lambda i: (0, 0)),
        ],
        out_specs=[
            pl.BlockSpec((rblk, out_dim), lambda i: (i, 0)),
            pl.BlockSpec((rblk, 1), lambda i: (i, 0)),
            pl.BlockSpec((rblk, 1), lambda i: (i, 0)),
        ],
        out_shape=[
            jax.ShapeDtypeStruct((n, out_dim), f32),
            jax.ShapeDtypeStruct((n, 1), f32),
            jax.ShapeDtypeStruct((n, 1), f32),
        ],
    )(h1o, W2, a2s, a2d)

    out = pl.pallas_call(
        functools.partial(_edge2_body, blk=blk, nblk=nblk),
        grid=(nblk,),
        in_specs=[
            pl.BlockSpec((1, 1, blk), lambda i: (i, 0, 0),
                         memory_space=pltpu.SMEM),
            pl.BlockSpec((1, 1, blk), lambda i: (i, 0, 0),
                         memory_space=pltpu.SMEM),
            pl.BlockSpec((n, out_dim), lambda i: (0, 0)),
            pl.BlockSpec((n, 1), lambda i: (0, 0)),
            pl.BlockSpec((n, 1), lambda i: (0, 0)),
            pl.BlockSpec((1, out_dim), lambda i: (0, 0)),
        ],
        out_specs=pl.BlockSpec((n, out_dim), lambda i: (0, 0)),
        out_shape=jax.ShapeDtypeStruct((n, out_dim), f32),
        scratch_shapes=[
            pltpu.VMEM((n, out_dim), f32),
            pltpu.VMEM((n, 1), f32),
        ],
    )(src3, dst3, h2, asrc2, adst2, b2.reshape(1, out_dim).astype(f32))

    return out


# speedup vs baseline: 6.2847x; 6.2847x over previous
import functools

import jax
import jax.numpy as jnp
from jax.experimental import pallas as pl
from jax.experimental.pallas import tpu as pltpu


def _lin1_body(x_ref, w_ref, ms_ref, md_ref, h_ref, asrc_ref, adst_ref):
    h = jnp.dot(x_ref[:], w_ref[:], preferred_element_type=jnp.float32)
    h_ref[:] = h
    asrc_ref[:] = jnp.dot(h, ms_ref[:], preferred_element_type=jnp.float32)
    adst_ref[:] = jnp.dot(h, md_ref[:], preferred_element_type=jnp.float32)


def _edge1_body(src_ref, dst_ref, h_ref, asrc_ref, adst_ref, sel_ref,
                acc_ref, den_ref, *, blk):
    pid = pl.program_id(0)

    @pl.when(pid == 0)
    def _():
        acc_ref[:] = jnp.zeros_like(acc_ref)
        den_ref[:] = jnp.zeros_like(den_ref)

    def body(j, carry):
        s = src_ref[0, 0, j]
        d = dst_ref[0, 0, j]
        v = asrc_ref[pl.ds(s, 1), :] + adst_ref[pl.ds(d, 1), :]
        v = jnp.where(v > 0, v, 0.2 * v)
        w = jnp.exp(v)
        den_ref[pl.ds(d, 1), :] += w
        wf = jnp.dot(w, sel_ref[:], preferred_element_type=jnp.float32)
        acc_ref[pl.ds(d, 1), :] += h_ref[pl.ds(s, 1), :] * wf
        return carry

    jax.lax.fori_loop(0, blk, body, 0, unroll=8)


def _norm1_body(acc_ref, den_ref, b_ref, sel_ref, o_ref):
    denf = jnp.dot(den_ref[:], sel_ref[:], preferred_element_type=jnp.float32)
    z = acc_ref[:] / denf + b_ref[:]
    o_ref[:] = jnp.where(z > 0, z, jnp.exp(z) - 1.0)


def _lin2_body(h_ref, w_ref, as_ref, ad_ref, h2_ref, asrc_ref, adst_ref):
    h2 = jnp.dot(h_ref[:], w_ref[:], preferred_element_type=jnp.float32)
    h2_ref[:] = h2
    asrc_ref[:] = jnp.dot(h2, as_ref[:], preferred_element_type=jnp.float32)
    adst_ref[:] = jnp.dot(h2, ad_ref[:], preferred_element_type=jnp.float32)


def _edge2_body(src_ref, dst_ref, h_ref, asrc_ref, adst_ref, b_ref, o_ref,
                acc_ref, den_ref, *, blk, nblk):
    pid = pl.program_id(0)

    @pl.when(pid == 0)
    def _():
        acc_ref[:] = jnp.zeros_like(acc_ref)
        den_ref[:] = jnp.zeros_like(den_ref)

    def body(j, carry):
        s = src_ref[0, 0, j]
        d = dst_ref[0, 0, j]
        v = asrc_ref[pl.ds(s, 1), :] + adst_ref[pl.ds(d, 1), :]
        v = jnp.where(v > 0, v, 0.2 * v)
        w = jnp.exp(v)
        den_ref[pl.ds(d, 1), :] += w
        acc_ref[pl.ds(d, 1), :] += h_ref[pl.ds(s, 1), :] * w
        return carry

    jax.lax.fori_loop(0, blk, body, 0, unroll=8)

    @pl.when(pid == nblk - 1)
    def _():
        o_ref[:] = acc_ref[:] / den_ref[:] + b_ref[:]


def kernel(x, edge_index, W1, a_src1, a_dst1, b1, W2, a_src2, a_dst2, b2):
    n, d_in = x.shape
    e = edge_index.shape[1]
    heads, ch = a_src1.shape[1], a_src1.shape[2]
    hc = heads * ch
    out_dim = W2.shape[1]
    f32 = jnp.float32

    loops = jnp.arange(n, dtype=edge_index.dtype)
    ei = jnp.concatenate([edge_index, jnp.stack([loops, loops])], axis=1)
    et = e + n
    blk = next(b for b in range(2048, 0, -1) if et % b == 0)
    nblk = et // blk
    src3 = ei[0].reshape(nblk, 1, blk)
    dst3 = ei[1].reshape(nblk, 1, blk)

    rows = jnp.arange(hc)
    cols = rows // ch
    ms1 = jnp.zeros((hc, heads), f32).at[rows, cols].set(a_src1.reshape(-1))
    md1 = jnp.zeros((hc, heads), f32).at[rows, cols].set(a_dst1.reshape(-1))
    sel = jnp.zeros((heads, hc), f32).at[cols, rows].set(1.0)
    a2s = a_src2.reshape(out_dim, 1).astype(f32)
    a2d = a_dst2.reshape(out_dim, 1).astype(f32)

    rblk = 1000 if n % 1000 == 0 else n
    nrb = n // rblk

    h1, asrc1, adst1 = pl.pallas_call(
        _lin1_body,
        grid=(nrb,),
        in_specs=[
            pl.BlockSpec((rblk, d_in), lambda i: (i, 0)),
            pl.BlockSpec((d_in, hc), lambda i: (0, 0)),
            pl.BlockSpec((hc, heads), lambda i: (0, 0)),
            pl.BlockSpec((hc, heads), lambda i: (0, 0)),
        ],
        out_specs=[
            pl.BlockSpec((rblk, hc), lambda i: (i, 0)),
            pl.BlockSpec((rblk, heads), lambda i: (i, 0)),
            pl.BlockSpec((rblk, heads), lambda i: (i, 0)),
        ],
        out_shape=[
            jax.ShapeDtypeStruct((n, hc), f32),
            jax.ShapeDtypeStruct((n, heads), f32),
            jax.ShapeDtypeStruct((n, heads), f32),
        ],
    )(x, W1, ms1, md1)

    acc1, den1 = pl.pallas_call(
        functools.partial(_edge1_body, blk=blk),
        grid=(nblk,),
        in_specs=[
            pl.BlockSpec((1, 1, blk), lambda i: (i, 0, 0),
                         memory_space=pltpu.SMEM),
            pl.BlockSpec((1, 1, blk), lambda i: (i, 0, 0),
                         memory_space=pltpu.SMEM),
            pl.BlockSpec((n, hc), lambda i: (0, 0)),
            pl.BlockSpec((n, heads), lambda i: (0, 0)),
            pl.BlockSpec((n, heads), lambda i: (0, 0)),
            pl.BlockSpec((heads, hc), lambda i: (0, 0)),
        ],
        out_specs=[
            pl.BlockSpec((n, hc), lambda i: (0, 0)),
            pl.BlockSpec((n, heads), lambda i: (0, 0)),
        ],
        out_shape=[
            jax.ShapeDtypeStruct((n, hc), f32),
            jax.ShapeDtypeStruct((n, heads), f32),
        ],
    )(src3, dst3, h1, asrc1, adst1, sel)

    h1o = pl.pallas_call(
        _norm1_body,
        grid=(nrb,),
        in_specs=[
            pl.BlockSpec((rblk, hc), lambda i: (i, 0)),
            pl.BlockSpec((rblk, heads), lambda i: (i, 0)),
            pl.BlockSpec((1, hc), lambda i: (0, 0)),
            pl.BlockSpec((heads, hc), lambda i: (0, 0)),
        ],
        out_specs=pl.BlockSpec((rblk, hc), lambda i: (i, 0)),
        out_shape=jax.ShapeDtypeStruct((n, hc), f32),
    )(acc1, den1, b1.reshape(1, hc).astype(f32), sel)

    h2, asrc2, adst2 = pl.pallas_call(
        _lin2_body,
        grid=(nrb,),
        in_specs=[
            pl.BlockSpec((rblk, hc), lambda i: (i, 0)),
            pl.BlockSpec((hc, out_dim), lambda i: (0, 0)),
            pl.BlockSpec((out_dim, 1), lambda i: (0, 0)),
            pl.BlockSpec((out_dim, 1), lambda i: (0, 0)),
        ],
        out_specs=[
            pl.BlockSpec((rblk, out_dim), lambda i: (i, 0)),
            pl.BlockSpec((rblk, 1), lambda i: (i, 0)),
            pl.BlockSpec((rblk, 1), lambda i: (i, 0)),
        ],
        out_shape=[
            jax.ShapeDtypeStruct((n, out_dim), f32),
            jax.ShapeDtypeStruct((n, 1), f32),
            jax.ShapeDtypeStruct((n, 1), f32),
        ],
    )(h1o, W2, a2s, a2d)

    out = pl.pallas_call(
        functools.partial(_edge2_body, blk=blk, nblk=nblk),
        grid=(nblk,),
        in_specs=[
            pl.BlockSpec((1, 1, blk), lambda i: (i, 0, 0),
                         memory_space=pltpu.SMEM),
            pl.BlockSpec((1, 1, blk), lambda i: (i, 0, 0),
                         memory_space=pltpu.SMEM),
            pl.BlockSpec((n, out_dim), lambda i: (0, 0)),
            pl.BlockSpec((n, 1), lambda i: (0, 0)),
            pl.BlockSpec((n, 1), lambda i: (0, 0)),
            pl.BlockSpec((1, out_dim), lambda i: (0, 0)),
        ],
        out_specs=pl.BlockSpec((n, out_dim), lambda i: (0, 0)),
        out_shape=jax.ShapeDtypeStruct((n, out_dim), f32),
        scratch_shapes=[
            pltpu.VMEM((n, out_dim), f32),
            pltpu.VMEM((n, 1), f32),
        ],
    )(src3, dst3, h2, asrc2, adst2, b2.reshape(1, out_dim).astype(f32))

    return out


# speedup vs baseline: 9.9467x; 1.5827x over previous
import functools

import jax
import jax.numpy as jnp
from jax.experimental import pallas as pl
from jax.experimental.pallas import tpu as pltpu


def _lin1_body(x_ref, w_ref, ms_ref, md_ref, h_ref, asrc_ref, adst_ref):
    h = jnp.dot(x_ref[:], w_ref[:], preferred_element_type=jnp.float32)
    h_ref[:] = h
    asrc_ref[:] = jnp.dot(h, ms_ref[:], preferred_element_type=jnp.float32)
    adst_ref[:] = jnp.dot(h, md_ref[:], preferred_element_type=jnp.float32)


def _edge1_body(src_ref, dst_ref, h_ref, asrc_ref, adst_ref, sel_ref,
                acc_ref, den_ref, *, blk):
    pid = pl.program_id(0)

    @pl.when(pid == 0)
    def _():
        acc_ref[:] = jnp.zeros_like(acc_ref)
        den_ref[:] = jnp.zeros_like(den_ref)

    def body(j, carry):
        s = src_ref[0, 0, j]
        d = dst_ref[0, 0, j]
        v = asrc_ref[pl.ds(s, 1), :] + adst_ref[pl.ds(d, 1), :]
        v = jnp.where(v > 0, v, 0.2 * v)
        w = jnp.exp(v)
        den_ref[pl.ds(d, 1), :] += w
        wf = jnp.dot(w, sel_ref[:], preferred_element_type=jnp.float32)
        acc_ref[pl.ds(d, 1), :] += h_ref[pl.ds(s, 1), :] * wf
        return carry

    jax.lax.fori_loop(0, blk, body, 0, unroll=16)


def _norm1_body(acc_ref, den_ref, b_ref, sel_ref, o_ref):
    denf = jnp.dot(den_ref[:], sel_ref[:], preferred_element_type=jnp.float32)
    z = acc_ref[:] / denf + b_ref[:]
    o_ref[:] = jnp.where(z > 0, z, jnp.exp(z) - 1.0)


def _lin2_body(h_ref, w_ref, as_ref, ad_ref, h2_ref, asrc_ref, adst_ref):
    h2 = jnp.dot(h_ref[:], w_ref[:], preferred_element_type=jnp.float32)
    h2_ref[:] = h2
    asrc_ref[:] = jnp.dot(h2, as_ref[:], preferred_element_type=jnp.float32)
    adst_ref[:] = jnp.dot(h2, ad_ref[:], preferred_element_type=jnp.float32)


def _edge2_body(src_ref, dst_ref, h_ref, asrc_ref, adst_ref, b_ref, o_ref,
                acc_ref, den_ref, *, blk, nblk):
    pid = pl.program_id(0)

    @pl.when(pid == 0)
    def _():
        acc_ref[:] = jnp.zeros_like(acc_ref)
        den_ref[:] = jnp.zeros_like(den_ref)

    def body(j, carry):
        s = src_ref[0, 0, j]
        d = dst_ref[0, 0, j]
        v = asrc_ref[pl.ds(s, 1), :] + adst_ref[pl.ds(d, 1), :]
        v = jnp.where(v > 0, v, 0.2 * v)
        w = jnp.exp(v)
        den_ref[pl.ds(d, 1), :] += w
        acc_ref[pl.ds(d, 1), :] += h_ref[pl.ds(s, 1), :] * w
        return carry

    jax.lax.fori_loop(0, blk, body, 0, unroll=16)

    @pl.when(pid == nblk - 1)
    def _():
        o_ref[:] = acc_ref[:] / den_ref[:] + b_ref[:]


def kernel(x, edge_index, W1, a_src1, a_dst1, b1, W2, a_src2, a_dst2, b2):
    n, d_in = x.shape
    e = edge_index.shape[1]
    heads, ch = a_src1.shape[1], a_src1.shape[2]
    hc = heads * ch
    out_dim = W2.shape[1]
    f32 = jnp.float32

    loops = jnp.arange(n, dtype=edge_index.dtype)
    ei = jnp.concatenate([edge_index, jnp.stack([loops, loops])], axis=1)
    et = e + n
    blk = next(b for b in range(2048, 0, -1) if et % b == 0)
    nblk = et // blk
    src3 = ei[0].reshape(nblk, 1, blk)
    dst3 = ei[1].reshape(nblk, 1, blk)

    rows = jnp.arange(hc)
    cols = rows // ch
    ms1 = jnp.zeros((hc, heads), f32).at[rows, cols].set(a_src1.reshape(-1))
    md1 = jnp.zeros((hc, heads), f32).at[rows, cols].set(a_dst1.reshape(-1))
    sel = jnp.zeros((heads, hc), f32).at[cols, rows].set(1.0)
    a2s = a_src2.reshape(out_dim, 1).astype(f32)
    a2d = a_dst2.reshape(out_dim, 1).astype(f32)

    rblk = 1000 if n % 1000 == 0 else n
    nrb = n // rblk

    h1, asrc1, adst1 = pl.pallas_call(
        _lin1_body,
        grid=(nrb,),
        in_specs=[
            pl.BlockSpec((rblk, d_in), lambda i: (i, 0)),
            pl.BlockSpec((d_in, hc), lambda i: (0, 0)),
            pl.BlockSpec((hc, heads), lambda i: (0, 0)),
            pl.BlockSpec((hc, heads), lambda i: (0, 0)),
        ],
        out_specs=[
            pl.BlockSpec((rblk, hc), lambda i: (i, 0)),
            pl.BlockSpec((rblk, heads), lambda i: (i, 0)),
            pl.BlockSpec((rblk, heads), lambda i: (i, 0)),
        ],
        out_shape=[
            jax.ShapeDtypeStruct((n, hc), f32),
            jax.ShapeDtypeStruct((n, heads), f32),
            jax.ShapeDtypeStruct((n, heads), f32),
        ],
    )(x, W1, ms1, md1)

    acc1, den1 = pl.pallas_call(
        functools.partial(_edge1_body, blk=blk),
        grid=(nblk,),
        in_specs=[
            pl.BlockSpec((1, 1, blk), lambda i: (i, 0, 0),
                         memory_space=pltpu.SMEM),
            pl.BlockSpec((1, 1, blk), lambda i: (i, 0, 0),
                         memory_space=pltpu.SMEM),
            pl.BlockSpec((n, hc), lambda i: (0, 0)),
            pl.BlockSpec((n, heads), lambda i: (0, 0)),
            pl.BlockSpec((n, heads), lambda i: (0, 0)),
            pl.BlockSpec((heads, hc), lambda i: (0, 0)),
        ],
        out_specs=[
            pl.BlockSpec((n, hc), lambda i: (0, 0)),
            pl.BlockSpec((n, heads), lambda i: (0, 0)),
        ],
        out_shape=[
            jax.ShapeDtypeStruct((n, hc), f32),
            jax.ShapeDtypeStruct((n, heads), f32),
        ],
    )(src3, dst3, h1, asrc1, adst1, sel)

    h1o = pl.pallas_call(
        _norm1_body,
        grid=(nrb,),
        in_specs=[
            pl.BlockSpec((rblk, hc), lambda i: (i, 0)),
            pl.BlockSpec((rblk, heads), lambda i: (i, 0)),
            pl.BlockSpec((1, hc), lambda i: (0, 0)),
            pl.BlockSpec((heads, hc), lambda i: (0, 0)),
        ],
        out_specs=pl.BlockSpec((rblk, hc), lambda i: (i, 0)),
        out_shape=jax.ShapeDtypeStruct((n, hc), f32),
    )(acc1, den1, b1.reshape(1, hc).astype(f32), sel)

    h2, asrc2, adst2 = pl.pallas_call(
        _lin2_body,
        grid=(nrb,),
        in_specs=[
            pl.BlockSpec((rblk, hc), lambda i: (i, 0)),
            pl.BlockSpec((hc, out_dim), lambda i: (0, 0)),
            pl.BlockSpec((out_dim, 1), lambda i: (0, 0)),
            pl.BlockSpec((out_dim, 1), lambda i: (0, 0)),
        ],
        out_specs=[
            pl.BlockSpec((rblk, out_dim), lambda i: (i, 0)),
            pl.BlockSpec((rblk, 1), lambda i: (i, 0)),
            pl.BlockSpec((rblk, 1), lambda i: (i, 0)),
        ],
        out_shape=[
            jax.ShapeDtypeStruct((n, out_dim), f32),
            jax.ShapeDtypeStruct((n, 1), f32),
            jax.ShapeDtypeStruct((n, 1), f32),
        ],
    )(h1o, W2, a2s, a2d)

    out = pl.pallas_call(
        functools.partial(_edge2_body, blk=blk, nblk=nblk),
        grid=(nblk,),
        in_specs=[
            pl.BlockSpec((1, 1, blk), lambda i: (i, 0, 0),
                         memory_space=pltpu.SMEM),
            pl.BlockSpec((1, 1, blk), lambda i: (i, 0, 0),
                         memory_space=pltpu.SMEM),
            pl.BlockSpec((n, out_dim), lambda i: (0, 0)),
            pl.BlockSpec((n, 1), lambda i: (0, 0)),
            pl.BlockSpec((n, 1), lambda i: (0, 0)),
            pl.BlockSpec((1, out_dim), lambda i: (0, 0)),
        ],
        out_specs=pl.BlockSpec((n, out_dim), lambda i: (0, 0)),
        out_shape=jax.ShapeDtypeStruct((n, out_dim), f32),
        scratch_shapes=[
            pltpu.VMEM((n, out_dim), f32),
            pltpu.VMEM((n, 1), f32),
        ],
    )(src3, dst3, h2, asrc2, adst2, b2.reshape(1, out_dim).astype(f32))

    return out


# speedup vs baseline: 12.6393x; 1.2707x over previous
import functools

import jax
import jax.numpy as jnp
from jax.experimental import pallas as pl
from jax.experimental.pallas import tpu as pltpu


def _lin1_body(x_ref, w_ref, ms_ref, md_ref, h_ref, asrc_ref, adst_ref):
    h = jnp.dot(x_ref[:], w_ref[:], preferred_element_type=jnp.float32)
    h_ref[:] = h
    asrc_ref[:] = jnp.dot(h, ms_ref[:], preferred_element_type=jnp.float32)
    adst_ref[:] = jnp.dot(h, md_ref[:], preferred_element_type=jnp.float32)


def _edge1_body(src_ref, dst_ref, h_ref, asrc_ref, adst_ref, sel_ref,
                acc_ref, den_ref, *, blk):
    pid = pl.program_id(0)

    @pl.when(pid == 0)
    def _():
        acc_ref[:] = jnp.zeros_like(acc_ref)
        den_ref[:] = jnp.zeros_like(den_ref)

    def body(j, carry):
        s = src_ref[0, 0, j]
        d = dst_ref[0, 0, j]
        v = asrc_ref[pl.ds(s, 1), :] + adst_ref[pl.ds(d, 1), :]
        v = jnp.where(v > 0, v, 0.2 * v)
        w = jnp.exp(v)
        den_ref[pl.ds(d, 1), :] += w
        wf = jnp.dot(w, sel_ref[:], preferred_element_type=jnp.float32)
        acc_ref[pl.ds(d, 1), :] += h_ref[pl.ds(s, 1), :] * wf
        return carry

    jax.lax.fori_loop(0, blk, body, 0, unroll=32)


def _norm1_body(acc_ref, den_ref, b_ref, sel_ref, o_ref):
    denf = jnp.dot(den_ref[:], sel_ref[:], preferred_element_type=jnp.float32)
    z = acc_ref[:] / denf + b_ref[:]
    o_ref[:] = jnp.where(z > 0, z, jnp.exp(z) - 1.0)


def _lin2_body(h_ref, w_ref, as_ref, ad_ref, h2_ref, asrc_ref, adst_ref):
    h2 = jnp.dot(h_ref[:], w_ref[:], preferred_element_type=jnp.float32)
    h2_ref[:] = h2
    asrc_ref[:] = jnp.dot(h2, as_ref[:], preferred_element_type=jnp.float32)
    adst_ref[:] = jnp.dot(h2, ad_ref[:], preferred_element_type=jnp.float32)


def _edge2_body(src_ref, dst_ref, h_ref, asrc_ref, adst_ref, b_ref, o_ref,
                acc_ref, den_ref, *, blk, nblk):
    pid = pl.program_id(0)

    @pl.when(pid == 0)
    def _():
        acc_ref[:] = jnp.zeros_like(acc_ref)
        den_ref[:] = jnp.zeros_like(den_ref)

    def body(j, carry):
        s = src_ref[0, 0, j]
        d = dst_ref[0, 0, j]
        v = asrc_ref[pl.ds(s, 1), :] + adst_ref[pl.ds(d, 1), :]
        v = jnp.where(v > 0, v, 0.2 * v)
        w = jnp.exp(v)
        den_ref[pl.ds(d, 1), :] += w
        acc_ref[pl.ds(d, 1), :] += h_ref[pl.ds(s, 1), :] * w
        return carry

    jax.lax.fori_loop(0, blk, body, 0, unroll=32)

    @pl.when(pid == nblk - 1)
    def _():
        o_ref[:] = acc_ref[:] / den_ref[:] + b_ref[:]


def kernel(x, edge_index, W1, a_src1, a_dst1, b1, W2, a_src2, a_dst2, b2):
    n, d_in = x.shape
    e = edge_index.shape[1]
    heads, ch = a_src1.shape[1], a_src1.shape[2]
    hc = heads * ch
    out_dim = W2.shape[1]
    f32 = jnp.float32

    loops = jnp.arange(n, dtype=edge_index.dtype)
    ei = jnp.concatenate([edge_index, jnp.stack([loops, loops])], axis=1)
    et = e + n
    blk = next(b for b in range(2048, 0, -1) if et % b == 0)
    nblk = et // blk
    src3 = ei[0].reshape(nblk, 1, blk)
    dst3 = ei[1].reshape(nblk, 1, blk)

    rows = jnp.arange(hc)
    cols = rows // ch
    ms1 = jnp.zeros((hc, heads), f32).at[rows, cols].set(a_src1.reshape(-1))
    md1 = jnp.zeros((hc, heads), f32).at[rows, cols].set(a_dst1.reshape(-1))
    sel = jnp.zeros((heads, hc), f32).at[cols, rows].set(1.0)
    a2s = a_src2.reshape(out_dim, 1).astype(f32)
    a2d = a_dst2.reshape(out_dim, 1).astype(f32)

    rblk = 1000 if n % 1000 == 0 else n
    nrb = n // rblk

    h1, asrc1, adst1 = pl.pallas_call(
        _lin1_body,
        grid=(nrb,),
        in_specs=[
            pl.BlockSpec((rblk, d_in), lambda i: (i, 0)),
            pl.BlockSpec((d_in, hc), lambda i: (0, 0)),
            pl.BlockSpec((hc, heads), lambda i: (0, 0)),
            pl.BlockSpec((hc, heads), lambda i: (0, 0)),
        ],
        out_specs=[
            pl.BlockSpec((rblk, hc), lambda i: (i, 0)),
            pl.BlockSpec((rblk, heads), lambda i: (i, 0)),
            pl.BlockSpec((rblk, heads), lambda i: (i, 0)),
        ],
        out_shape=[
            jax.ShapeDtypeStruct((n, hc), f32),
            jax.ShapeDtypeStruct((n, heads), f32),
            jax.ShapeDtypeStruct((n, heads), f32),
        ],
    )(x, W1, ms1, md1)

    acc1, den1 = pl.pallas_call(
        functools.partial(_edge1_body, blk=blk),
        grid=(nblk,),
        in_specs=[
            pl.BlockSpec((1, 1, blk), lambda i: (i, 0, 0),
                         memory_space=pltpu.SMEM),
            pl.BlockSpec((1, 1, blk), lambda i: (i, 0, 0),
                         memory_space=pltpu.SMEM),
            pl.BlockSpec((n, hc), lambda i: (0, 0)),
            pl.BlockSpec((n, heads), lambda i: (0, 0)),
            pl.BlockSpec((n, heads), lambda i: (0, 0)),
            pl.BlockSpec((heads, hc), lambda i: (0, 0)),
        ],
        out_specs=[
            pl.BlockSpec((n, hc), lambda i: (0, 0)),
            pl.BlockSpec((n, heads), lambda i: (0, 0)),
        ],
        out_shape=[
            jax.ShapeDtypeStruct((n, hc), f32),
            jax.ShapeDtypeStruct((n, heads), f32),
        ],
    )(src3, dst3, h1, asrc1, adst1, sel)

    h1o = pl.pallas_call(
        _norm1_body,
        grid=(nrb,),
        in_specs=[
            pl.BlockSpec((rblk, hc), lambda i: (i, 0)),
            pl.BlockSpec((rblk, heads), lambda i: (i, 0)),
            pl.BlockSpec((1, hc), lambda i: (0, 0)),
            pl.BlockSpec((heads, hc), lambda i: (0, 0)),
        ],
        out_specs=pl.BlockSpec((rblk, hc), lambda i: (i, 0)),
        out_shape=jax.ShapeDtypeStruct((n, hc), f32),
    )(acc1, den1, b1.reshape(1, hc).astype(f32), sel)

    h2, asrc2, adst2 = pl.pallas_call(
        _lin2_body,
        grid=(nrb,),
        in_specs=[
            pl.BlockSpec((rblk, hc), lambda i: (i, 0)),
            pl.BlockSpec((hc, out_dim), lambda i: (0, 0)),
            pl.BlockSpec((out_dim, 1), lambda i: (0, 0)),
            pl.BlockSpec((out_dim, 1), lambda i: (0, 0)),
        ],
        out_specs=[
            pl.BlockSpec((rblk, out_dim), lambda i: (i, 0)),
            pl.BlockSpec((rblk, 1), lambda i: (i, 0)),
            pl.BlockSpec((rblk, 1), lambda i: (i, 0)),
        ],
        out_shape=[
            jax.ShapeDtypeStruct((n, out_dim), f32),
            jax.ShapeDtypeStruct((n, 1), f32),
            jax.ShapeDtypeStruct((n, 1), f32),
        ],
    )(h1o, W2, a2s, a2d)

    out = pl.pallas_call(
        functools.partial(_edge2_body, blk=blk, nblk=nblk),
        grid=(nblk,),
        in_specs=[
            pl.BlockSpec((1, 1, blk), lambda i: (i, 0, 0),
                         memory_space=pltpu.SMEM),
            pl.BlockSpec((1, 1, blk), lambda i: (i, 0, 0),
                         memory_space=pltpu.SMEM),
            pl.BlockSpec((n, out_dim), lambda i: (0, 0)),
            pl.BlockSpec((n, 1), lambda i: (0, 0)),
            pl.BlockSpec((n, 1), lambda i: (0, 0)),
            pl.BlockSpec((1, out_dim), lambda i: (0, 0)),
        ],
        out_specs=pl.BlockSpec((n, out_dim), lambda i: (0, 0)),
        out_shape=jax.ShapeDtypeStruct((n, out_dim), f32),
        scratch_shapes=[
            pltpu.VMEM((n, out_dim), f32),
            pltpu.VMEM((n, 1), f32),
        ],
    )(src3, dst3, h2, asrc2, adst2, b2.reshape(1, out_dim).astype(f32))

    return out


# speedup vs baseline: 14.6788x; 1.1614x over previous
import functools

import jax
import jax.numpy as jnp
from jax.experimental import pallas as pl
from jax.experimental.pallas import tpu as pltpu


def _lin1_body(x_ref, w_ref, ms_ref, md_ref, h_ref, asrc_ref, adst_ref):
    h = jnp.dot(x_ref[:], w_ref[:], preferred_element_type=jnp.float32)
    h_ref[:] = h
    asrc_ref[:] = jnp.dot(h, ms_ref[:], preferred_element_type=jnp.float32)
    adst_ref[:] = jnp.dot(h, md_ref[:], preferred_element_type=jnp.float32)


def _edge1_body(src_ref, dst_ref, h_ref, asrc_ref, adst_ref, sel_ref,
                acc_ref, den_ref, *, blk):
    pid = pl.program_id(0)

    @pl.when(pid == 0)
    def _():
        acc_ref[:] = jnp.zeros_like(acc_ref)
        den_ref[:] = jnp.zeros_like(den_ref)

    def body(j, carry):
        s = src_ref[0, 0, j]
        d = dst_ref[0, 0, j]
        v = asrc_ref[pl.ds(s, 1), :] + adst_ref[pl.ds(d, 1), :]
        v = jnp.where(v > 0, v, 0.2 * v)
        w = jnp.exp(v)
        den_ref[pl.ds(d, 1), :] += w
        wf = jnp.dot(w, sel_ref[:], preferred_element_type=jnp.float32)
        acc_ref[pl.ds(d, 1), :] += h_ref[pl.ds(s, 1), :] * wf
        return carry

    jax.lax.fori_loop(0, blk, body, 0, unroll=64)


def _norm1_body(acc_ref, den_ref, b_ref, sel_ref, o_ref):
    denf = jnp.dot(den_ref[:], sel_ref[:], preferred_element_type=jnp.float32)
    z = acc_ref[:] / denf + b_ref[:]
    o_ref[:] = jnp.where(z > 0, z, jnp.exp(z) - 1.0)


def _lin2_body(h_ref, w_ref, as_ref, ad_ref, h2_ref, asrc_ref, adst_ref):
    h2 = jnp.dot(h_ref[:], w_ref[:], preferred_element_type=jnp.float32)
    h2_ref[:] = h2
    asrc_ref[:] = jnp.dot(h2, as_ref[:], preferred_element_type=jnp.float32)
    adst_ref[:] = jnp.dot(h2, ad_ref[:], preferred_element_type=jnp.float32)


def _edge2_body(src_ref, dst_ref, h_ref, asrc_ref, adst_ref, b_ref, o_ref,
                acc_ref, den_ref, *, blk, nblk):
    pid = pl.program_id(0)

    @pl.when(pid == 0)
    def _():
        acc_ref[:] = jnp.zeros_like(acc_ref)
        den_ref[:] = jnp.zeros_like(den_ref)

    def body(j, carry):
        s = src_ref[0, 0, j]
        d = dst_ref[0, 0, j]
        v = asrc_ref[pl.ds(s, 1), :] + adst_ref[pl.ds(d, 1), :]
        v = jnp.where(v > 0, v, 0.2 * v)
        w = jnp.exp(v)
        den_ref[pl.ds(d, 1), :] += w
        acc_ref[pl.ds(d, 1), :] += h_ref[pl.ds(s, 1), :] * w
        return carry

    jax.lax.fori_loop(0, blk, body, 0, unroll=64)

    @pl.when(pid == nblk - 1)
    def _():
        o_ref[:] = acc_ref[:] / den_ref[:] + b_ref[:]


def kernel(x, edge_index, W1, a_src1, a_dst1, b1, W2, a_src2, a_dst2, b2):
    n, d_in = x.shape
    e = edge_index.shape[1]
    heads, ch = a_src1.shape[1], a_src1.shape[2]
    hc = heads * ch
    out_dim = W2.shape[1]
    f32 = jnp.float32

    loops = jnp.arange(n, dtype=edge_index.dtype)
    ei = jnp.concatenate([edge_index, jnp.stack([loops, loops])], axis=1)
    et = e + n
    blk = next(b for b in range(2048, 0, -1) if et % b == 0)
    nblk = et // blk
    src3 = ei[0].reshape(nblk, 1, blk)
    dst3 = ei[1].reshape(nblk, 1, blk)

    rows = jnp.arange(hc)
    cols = rows // ch
    ms1 = jnp.zeros((hc, heads), f32).at[rows, cols].set(a_src1.reshape(-1))
    md1 = jnp.zeros((hc, heads), f32).at[rows, cols].set(a_dst1.reshape(-1))
    sel = jnp.zeros((heads, hc), f32).at[cols, rows].set(1.0)
    a2s = a_src2.reshape(out_dim, 1).astype(f32)
    a2d = a_dst2.reshape(out_dim, 1).astype(f32)

    rblk = 1000 if n % 1000 == 0 else n
    nrb = n // rblk

    h1, asrc1, adst1 = pl.pallas_call(
        _lin1_body,
        grid=(nrb,),
        in_specs=[
            pl.BlockSpec((rblk, d_in), lambda i: (i, 0)),
            pl.BlockSpec((d_in, hc), lambda i: (0, 0)),
            pl.BlockSpec((hc, heads), lambda i: (0, 0)),
            pl.BlockSpec((hc, heads), lambda i: (0, 0)),
        ],
        out_specs=[
            pl.BlockSpec((rblk, hc), lambda i: (i, 0)),
            pl.BlockSpec((rblk, heads), lambda i: (i, 0)),
            pl.BlockSpec((rblk, heads), lambda i: (i, 0)),
        ],
        out_shape=[
            jax.ShapeDtypeStruct((n, hc), f32),
            jax.ShapeDtypeStruct((n, heads), f32),
            jax.ShapeDtypeStruct((n, heads), f32),
        ],
    )(x, W1, ms1, md1)

    acc1, den1 = pl.pallas_call(
        functools.partial(_edge1_body, blk=blk),
        grid=(nblk,),
        in_specs=[
            pl.BlockSpec((1, 1, blk), lambda i: (i, 0, 0),
                         memory_space=pltpu.SMEM),
            pl.BlockSpec((1, 1, blk), lambda i: (i, 0, 0),
                         memory_space=pltpu.SMEM),
            pl.BlockSpec((n, hc), lambda i: (0, 0)),
            pl.BlockSpec((n, heads), lambda i: (0, 0)),
            pl.BlockSpec((n, heads), lambda i: (0, 0)),
            pl.BlockSpec((heads, hc), lambda i: (0, 0)),
        ],
        out_specs=[
            pl.BlockSpec((n, hc), lambda i: (0, 0)),
            pl.BlockSpec((n, heads), lambda i: (0, 0)),
        ],
        out_shape=[
            jax.ShapeDtypeStruct((n, hc), f32),
            jax.ShapeDtypeStruct((n, heads), f32),
        ],
    )(src3, dst3, h1, asrc1, adst1, sel)

    h1o = pl.pallas_call(
        _norm1_body,
        grid=(nrb,),
        in_specs=[
            pl.BlockSpec((rblk, hc), lambda i: (i, 0)),
            pl.BlockSpec((rblk, heads), lambda i: (i, 0)),
            pl.BlockSpec((1, hc), lambda i: (0, 0)),
            pl.BlockSpec((heads, hc), lambda i: (0, 0)),
        ],
        out_specs=pl.BlockSpec((rblk, hc), lambda i: (i, 0)),
        out_shape=jax.ShapeDtypeStruct((n, hc), f32),
    )(acc1, den1, b1.reshape(1, hc).astype(f32), sel)

    h2, asrc2, adst2 = pl.pallas_call(
        _lin2_body,
        grid=(nrb,),
        in_specs=[
            pl.BlockSpec((rblk, hc), lambda i: (i, 0)),
            pl.BlockSpec((hc, out_dim), lambda i: (0, 0)),
            pl.BlockSpec((out_dim, 1), lambda i: (0, 0)),
            pl.BlockSpec((out_dim, 1), lambda i: (0, 0)),
        ],
        out_specs=[
            pl.BlockSpec((rblk, out_dim), lambda i: (i, 0)),
            pl.BlockSpec((rblk, 1), lambda i: (i, 0)),
            pl.BlockSpec((rblk, 1), lambda i: (i, 0)),
        ],
        out_shape=[
            jax.ShapeDtypeStruct((n, out_dim), f32),
            jax.ShapeDtypeStruct((n, 1), f32),
            jax.ShapeDtypeStruct((n, 1), f32),
        ],
    )(h1o, W2, a2s, a2d)

    out = pl.pallas_call(
        functools.partial(_edge2_body, blk=blk, nblk=nblk),
        grid=(nblk,),
        in_specs=[
            pl.BlockSpec((1, 1, blk), lambda i: (i, 0, 0),
                         memory_space=pltpu.SMEM),
            pl.BlockSpec((1, 1, blk), lambda i: (i, 0, 0),
                         memory_space=pltpu.SMEM),
            pl.BlockSpec((n, out_dim), lambda i: (0, 0)),
            pl.BlockSpec((n, 1), lambda i: (0, 0)),
            pl.BlockSpec((n, 1), lambda i: (0, 0)),
            pl.BlockSpec((1, out_dim), lambda i: (0, 0)),
        ],
        out_specs=pl.BlockSpec((n, out_dim), lambda i: (0, 0)),
        out_shape=jax.ShapeDtypeStruct((n, out_dim), f32),
        scratch_shapes=[
            pltpu.VMEM((n, out_dim), f32),
            pltpu.VMEM((n, 1), f32),
        ],
    )(src3, dst3, h2, asrc2, adst2, b2.reshape(1, out_dim).astype(f32))

    return out


# speedup vs baseline: 16.2891x; 1.1097x over previous
import functools

import jax
import jax.numpy as jnp
from jax.experimental import pallas as pl
from jax.experimental.pallas import tpu as pltpu


def _lin1_body(x_ref, w_ref, ms_ref, md_ref, h_ref, asrc_ref, adst_ref):
    h = jnp.dot(x_ref[:], w_ref[:], preferred_element_type=jnp.float32)
    h_ref[:] = h
    asrc_ref[:] = jnp.dot(h, ms_ref[:], preferred_element_type=jnp.float32)
    adst_ref[:] = jnp.dot(h, md_ref[:], preferred_element_type=jnp.float32)


def _edge1_body(src_ref, dst_ref, h_ref, asrc_ref, adst_ref, sel_ref,
                acc_ref, den_ref, *, blk):
    pid = pl.program_id(0)

    @pl.when(pid == 0)
    def _():
        acc_ref[:] = jnp.zeros_like(acc_ref)
        den_ref[:] = jnp.zeros_like(den_ref)

    def body(j, carry):
        s = src_ref[0, 0, j]
        d = dst_ref[0, 0, j]
        v = asrc_ref[pl.ds(s, 1), :] + adst_ref[pl.ds(d, 1), :]
        v = jnp.where(v > 0, v, 0.2 * v)
        w = jnp.exp(v)
        den_ref[pl.ds(d, 1), :] += w
        wf = jnp.dot(w, sel_ref[:], preferred_element_type=jnp.float32)
        acc_ref[pl.ds(d, 1), :] += h_ref[pl.ds(s, 1), :] * wf
        return carry

    jax.lax.fori_loop(0, blk, body, 0, unroll=125)


def _norm1_body(acc_ref, den_ref, b_ref, sel_ref, o_ref):
    denf = jnp.dot(den_ref[:], sel_ref[:], preferred_element_type=jnp.float32)
    z = acc_ref[:] / denf + b_ref[:]
    o_ref[:] = jnp.where(z > 0, z, jnp.exp(z) - 1.0)


def _lin2_body(h_ref, w_ref, as_ref, ad_ref, h2_ref, asrc_ref, adst_ref):
    h2 = jnp.dot(h_ref[:], w_ref[:], preferred_element_type=jnp.float32)
    h2_ref[:] = h2
    asrc_ref[:] = jnp.dot(h2, as_ref[:], preferred_element_type=jnp.float32)
    adst_ref[:] = jnp.dot(h2, ad_ref[:], preferred_element_type=jnp.float32)


def _edge2_body(src_ref, dst_ref, h_ref, asrc_ref, adst_ref, b_ref, o_ref,
                acc_ref, den_ref, *, blk, nblk):
    pid = pl.program_id(0)

    @pl.when(pid == 0)
    def _():
        acc_ref[:] = jnp.zeros_like(acc_ref)
        den_ref[:] = jnp.zeros_like(den_ref)

    def body(j, carry):
        s = src_ref[0, 0, j]
        d = dst_ref[0, 0, j]
        v = asrc_ref[pl.ds(s, 1), :] + adst_ref[pl.ds(d, 1), :]
        v = jnp.where(v > 0, v, 0.2 * v)
        w = jnp.exp(v)
        den_ref[pl.ds(d, 1), :] += w
        acc_ref[pl.ds(d, 1), :] += h_ref[pl.ds(s, 1), :] * w
        return carry

    jax.lax.fori_loop(0, blk, body, 0, unroll=125)

    @pl.when(pid == nblk - 1)
    def _():
        o_ref[:] = acc_ref[:] / den_ref[:] + b_ref[:]


def kernel(x, edge_index, W1, a_src1, a_dst1, b1, W2, a_src2, a_dst2, b2):
    n, d_in = x.shape
    e = edge_index.shape[1]
    heads, ch = a_src1.shape[1], a_src1.shape[2]
    hc = heads * ch
    out_dim = W2.shape[1]
    f32 = jnp.float32

    loops = jnp.arange(n, dtype=edge_index.dtype)
    ei = jnp.concatenate([edge_index, jnp.stack([loops, loops])], axis=1)
    et = e + n
    blk = next(b for b in range(2048, 0, -1) if et % b == 0)
    nblk = et // blk
    src3 = ei[0].reshape(nblk, 1, blk)
    dst3 = ei[1].reshape(nblk, 1, blk)

    rows = jnp.arange(hc)
    cols = rows // ch
    ms1 = jnp.zeros((hc, heads), f32).at[rows, cols].set(a_src1.reshape(-1))
    md1 = jnp.zeros((hc, heads), f32).at[rows, cols].set(a_dst1.reshape(-1))
    sel = jnp.zeros((heads, hc), f32).at[cols, rows].set(1.0)
    a2s = a_src2.reshape(out_dim, 1).astype(f32)
    a2d = a_dst2.reshape(out_dim, 1).astype(f32)

    rblk = 1000 if n % 1000 == 0 else n
    nrb = n // rblk

    h1, asrc1, adst1 = pl.pallas_call(
        _lin1_body,
        grid=(nrb,),
        in_specs=[
            pl.BlockSpec((rblk, d_in), lambda i: (i, 0)),
            pl.BlockSpec((d_in, hc), lambda i: (0, 0)),
            pl.BlockSpec((hc, heads), lambda i: (0, 0)),
            pl.BlockSpec((hc, heads), lambda i: (0, 0)),
        ],
        out_specs=[
            pl.BlockSpec((rblk, hc), lambda i: (i, 0)),
            pl.BlockSpec((rblk, heads), lambda i: (i, 0)),
            pl.BlockSpec((rblk, heads), lambda i: (i, 0)),
        ],
        out_shape=[
            jax.ShapeDtypeStruct((n, hc), f32),
            jax.ShapeDtypeStruct((n, heads), f32),
            jax.ShapeDtypeStruct((n, heads), f32),
        ],
    )(x, W1, ms1, md1)

    acc1, den1 = pl.pallas_call(
        functools.partial(_edge1_body, blk=blk),
        grid=(nblk,),
        in_specs=[
            pl.BlockSpec((1, 1, blk), lambda i: (i, 0, 0),
                         memory_space=pltpu.SMEM),
            pl.BlockSpec((1, 1, blk), lambda i: (i, 0, 0),
                         memory_space=pltpu.SMEM),
            pl.BlockSpec((n, hc), lambda i: (0, 0)),
            pl.BlockSpec((n, heads), lambda i: (0, 0)),
            pl.BlockSpec((n, heads), lambda i: (0, 0)),
            pl.BlockSpec((heads, hc), lambda i: (0, 0)),
        ],
        out_specs=[
            pl.BlockSpec((n, hc), lambda i: (0, 0)),
            pl.BlockSpec((n, heads), lambda i: (0, 0)),
        ],
        out_shape=[
            jax.ShapeDtypeStruct((n, hc), f32),
            jax.ShapeDtypeStruct((n, heads), f32),
        ],
    )(src3, dst3, h1, asrc1, adst1, sel)

    h1o = pl.pallas_call(
        _norm1_body,
        grid=(nrb,),
        in_specs=[
            pl.BlockSpec((rblk, hc), lambda i: (i, 0)),
            pl.BlockSpec((rblk, heads), lambda i: (i, 0)),
            pl.BlockSpec((1, hc), lambda i: (0, 0)),
            pl.BlockSpec((heads, hc), lambda i: (0, 0)),
        ],
        out_specs=pl.BlockSpec((rblk, hc), lambda i: (i, 0)),
        out_shape=jax.ShapeDtypeStruct((n, hc), f32),
    )(acc1, den1, b1.reshape(1, hc).astype(f32), sel)

    h2, asrc2, adst2 = pl.pallas_call(
        _lin2_body,
        grid=(nrb,),
        in_specs=[
            pl.BlockSpec((rblk, hc), lambda i: (i, 0)),
            pl.BlockSpec((hc, out_dim), lambda i: (0, 0)),
            pl.BlockSpec((out_dim, 1), lambda i: (0, 0)),
            pl.BlockSpec((out_dim, 1), lambda i: (0, 0)),
        ],
        out_specs=[
            pl.BlockSpec((rblk, out_dim), lambda i: (i, 0)),
            pl.BlockSpec((rblk, 1), lambda i: (i, 0)),
            pl.BlockSpec((rblk, 1), lambda i: (i, 0)),
        ],
        out_shape=[
            jax.ShapeDtypeStruct((n, out_dim), f32),
            jax.ShapeDtypeStruct((n, 1), f32),
            jax.ShapeDtypeStruct((n, 1), f32),
        ],
    )(h1o, W2, a2s, a2d)

    out = pl.pallas_call(
        functools.partial(_edge2_body, blk=blk, nblk=nblk),
        grid=(nblk,),
        in_specs=[
            pl.BlockSpec((1, 1, blk), lambda i: (i, 0, 0),
                         memory_space=pltpu.SMEM),
            pl.BlockSpec((1, 1, blk), lambda i: (i, 0, 0),
                         memory_space=pltpu.SMEM),
            pl.BlockSpec((n, out_dim), lambda i: (0, 0)),
            pl.BlockSpec((n, 1), lambda i: (0, 0)),
            pl.BlockSpec((n, 1), lambda i: (0, 0)),
            pl.BlockSpec((1, out_dim), lambda i: (0, 0)),
        ],
        out_specs=pl.BlockSpec((n, out_dim), lambda i: (0, 0)),
        out_shape=jax.ShapeDtypeStruct((n, out_dim), f32),
        scratch_shapes=[
            pltpu.VMEM((n, out_dim), f32),
            pltpu.VMEM((n, 1), f32),
        ],
    )(src3, dst3, h2, asrc2, adst2, b2.reshape(1, out_dim).astype(f32))

    return out
